# baseline (speedup 1.0000x reference)
import jax
import jax.numpy as jnp
from jax.experimental import pallas as pl
from jax.experimental.pallas import tpu as pltpu


_N_CHUNKS = 32


def _copy_kernel(in_ref, out_ref, sems):
    rows = in_ref.shape[0]
    chunk = rows // _N_CHUNKS
    copies = []
    for i in range(_N_CHUNKS):
        c = pltpu.make_async_copy(
            in_ref.at[pl.ds(i * chunk, chunk)],
            out_ref.at[pl.ds(i * chunk, chunk)],
            sems.at[i],
        )
        c.start()
        copies.append(c)
    for c in copies:
        c.wait()


def kernel(mps):
    n_mp, n, m = mps.shape
    flat = mps.reshape(n_mp * n, m)
    out = pl.pallas_call(
        _copy_kernel,
        in_specs=[pl.BlockSpec(memory_space=pl.ANY)],
        out_specs=pl.BlockSpec(memory_space=pl.ANY),
        scratch_shapes=[pltpu.SemaphoreType.DMA((_N_CHUNKS,))],
        out_shape=jax.ShapeDtypeStruct(flat.shape, flat.dtype),
    )(flat)
    return out.reshape(n_mp, n, m)


# speedup vs baseline: 48.2221x; 48.2221x over previous
import jax
import jax.numpy as jnp
from jax.experimental import pallas as pl
from jax.experimental.pallas import tpu as pltpu


_BLOCK_ROWS = 256


def _copy_kernel(in_ref, out_ref):
    out_ref[...] = in_ref[...]


def kernel(mps):
    n_mp, n, m = mps.shape
    flat = mps.reshape(n_mp * n, m)
    out = pl.pallas_call(
        _copy_kernel,
        grid=(flat.shape[0] // _BLOCK_ROWS,),
        in_specs=[pl.BlockSpec((_BLOCK_ROWS, m), lambda i: (i, 0))],
        out_specs=pl.BlockSpec((_BLOCK_ROWS, m), lambda i: (i, 0)),
        out_shape=jax.ShapeDtypeStruct(flat.shape, flat.dtype),
    )(flat)
    return out.reshape(n_mp, n, m)


# speedup vs baseline: 49.2500x; 1.0213x over previous
import jax
import jax.numpy as jnp
from jax.experimental import pallas as pl
from jax.experimental.pallas import tpu as pltpu


_BLOCK_ROWS = 512


def _copy_kernel(in_ref, out_ref):
    out_ref[...] = in_ref[...]


def kernel(mps):
    n_mp, n, m = mps.shape
    flat = mps.reshape(n_mp * n, m)
    out = pl.pallas_call(
        _copy_kernel,
        grid=(flat.shape[0] // _BLOCK_ROWS,),
        in_specs=[pl.BlockSpec((_BLOCK_ROWS, m), lambda i: (i, 0))],
        out_specs=pl.BlockSpec((_BLOCK_ROWS, m), lambda i: (i, 0)),
        out_shape=jax.ShapeDtypeStruct(flat.shape, flat.dtype),
    )(flat)
    return out.reshape(n_mp, n, m)
